# baseline (speedup 1.0000x reference)
import functools

import jax
import jax.numpy as jnp
from jax import lax
from jax.experimental import pallas as pl
from jax.experimental.pallas import tpu as pltpu
from jax.experimental.pallas import tpu_sc as plsc

N_NODES = 10000
N_PAD = 10240
N_EDGES = 160000
D = 256
H = D // 2
CHUNK = 128
N_TILES = 16
K_PER_TILE = 80
N_CHUNKS = N_TILES * K_PER_TILE
E_PADDED = N_CHUNKS * CHUNK
ROWS_PER_TILE = N_PAD // N_TILES
ZROW = 2 * N_NODES


def _sc_segment_sum(xh, src_idx, dst_idx, zeros):
    mesh = plsc.VectorSubcoreMesh(core_axis_name="c", subcore_axis_name="s")

    @functools.partial(
        pl.kernel,
        out_type=jax.ShapeDtypeStruct((2 * N_PAD, H), jnp.float32),
        mesh=mesh,
        scratch_types=[
            pltpu.VMEM((CHUNK,), jnp.int32),
            pltpu.VMEM((CHUNK,), jnp.int32),
            pltpu.VMEM((CHUNK,), jnp.int32),
            pltpu.VMEM((CHUNK,), jnp.int32),
            pltpu.VMEM((CHUNK, H), jnp.float32),
            pltpu.VMEM((CHUNK, H), jnp.float32),
            pltpu.VMEM_SHARED((N_PAD, H), jnp.float32),
        ]
        + [pltpu.SemaphoreType.DMA] * 4,
    )
    def sc_kernel(xh_hbm, src_hbm, dst_hbm, zeros_hbm, out_hbm,
                  src0, src1, dst0, dst1, rows0, rows1, acc_sh,
                  sg0, sg1, ss0, ss1):
        c = lax.axis_index("c")
        s = lax.axis_index("s")
        slots = ((src0, dst0, rows0, sg0, ss0), (src1, dst1, rows1, sg1, ss1))

        def idx_load(i, sl):
            src_v, dst_v = slots[sl][0], slots[sl][1]
            pltpu.sync_copy(src_hbm.at[c, i], src_v)
            pltpu.sync_copy(dst_hbm.at[i], dst_v)

        def g_start(sl):
            src_v, rows_v, sem = slots[sl][0], slots[sl][2], slots[sl][3]
            pltpu.async_copy(xh_hbm.at[src_v], rows_v, sem)

        def g_wait(sl):
            src_v, rows_v, sem = slots[sl][0], slots[sl][2], slots[sl][3]
            pltpu.make_async_copy(xh_hbm.at[src_v], rows_v, sem).wait()

        def s_start(sl):
            dst_v, rows_v, sem = slots[sl][1], slots[sl][2], slots[sl][4]
            pltpu.async_copy(rows_v, acc_sh.at[dst_v], sem, add=True)

        def s_wait(sl):
            dst_v, rows_v, sem = slots[sl][1], slots[sl][2], slots[sl][4]
            pltpu.make_async_copy(rows_v, acc_sh.at[dst_v], sem).wait()

        idx_load(s, 0)
        g_start(0)
        pltpu.sync_copy(zeros_hbm, rows1)
        for z in range(ROWS_PER_TILE // CHUNK):
            pltpu.sync_copy(
                rows1, acc_sh.at[pl.ds(s * ROWS_PER_TILE + z * CHUNK, CHUNK)])
        plsc.subcore_barrier()

        idx_load(N_TILES + s, 1)
        g_wait(0)
        g_start(1)
        s_start(0)

        @pl.loop(0, (K_PER_TILE - 2) // 2)
        def _(g):
            for half in range(2):
                k = 2 * g + 1 + half
                cur, nxt = (1, 0) if half == 0 else (0, 1)
                s_wait(nxt)
                idx_load((k + 1) * N_TILES + s, nxt)
                g_wait(cur)
                g_start(nxt)
                s_start(cur)

        s_wait(0)
        g_wait(1)
        s_start(1)
        s_wait(1)

        plsc.subcore_barrier()
        pltpu.sync_copy(
            acc_sh.at[pl.ds(s * ROWS_PER_TILE, ROWS_PER_TILE)],
            out_hbm.at[pl.ds(c * N_PAD + s * ROWS_PER_TILE, ROWS_PER_TILE)],
        )

    return sc_kernel(xh, src_idx, dst_idx, zeros)


def _tc_root_body(x_ref, wr_ref, b_ref, o_ref):
    o_ref[...] = jnp.dot(x_ref[...], wr_ref[...],
                         preferred_element_type=jnp.float32) + b_ref[...]


def _tc_root(x, wr, b2d):
    n = x.shape[0]
    blk = 1000
    return pl.pallas_call(
        _tc_root_body,
        grid=(n // blk,),
        in_specs=[
            pl.BlockSpec((blk, D), lambda i: (i, 0)),
            pl.BlockSpec((D, D), lambda i: (0, 0)),
            pl.BlockSpec((1, D), lambda i: (0, 0)),
        ],
        out_specs=pl.BlockSpec((blk, D), lambda i: (i, 0)),
        out_shape=jax.ShapeDtypeStruct((n, D), jnp.float32),
    )(x, wr, b2d)


def _tc_finish_body(a0_ref, a1_ref, z_ref, w0_ref, w1_ref, o_ref):
    y = jnp.dot(a0_ref[...], w0_ref[...], preferred_element_type=jnp.float32)
    y = y + jnp.dot(a1_ref[...], w1_ref[...], preferred_element_type=jnp.float32)
    y = y + z_ref[...]
    m = jnp.max(y, axis=-1, keepdims=True)
    t = y - m
    lse = jnp.log(jnp.sum(jnp.exp(t), axis=-1, keepdims=True))
    o_ref[...] = t - lse


def _tc_finish(agg0, agg1, z, w0, w1):
    n = z.shape[0]
    blk = 1000
    return pl.pallas_call(
        _tc_finish_body,
        grid=(n // blk,),
        in_specs=[
            pl.BlockSpec((blk, H), lambda i: (i, 0)),
            pl.BlockSpec((blk, H), lambda i: (i, 0)),
            pl.BlockSpec((blk, D), lambda i: (i, 0)),
            pl.BlockSpec((H, D), lambda i: (0, 0)),
            pl.BlockSpec((H, D), lambda i: (0, 0)),
        ],
        out_specs=pl.BlockSpec((blk, D), lambda i: (i, 0)),
        out_shape=jax.ShapeDtypeStruct((n, D), jnp.float32),
    )(agg0, agg1, z, w0, w1)


def kernel(x, edge_index, W_rel, W_root, b):
    src = edge_index[0]
    dst = edge_index[1]
    n_extra = E_PADDED - N_EDGES
    pad_src = jnp.full((n_extra,), ZROW, jnp.int32)
    src0 = jnp.concatenate([src, pad_src]).reshape(N_CHUNKS, CHUNK)
    src1 = jnp.concatenate([src + N_NODES, pad_src]).reshape(N_CHUNKS, CHUNK)
    dst_pad = jnp.concatenate(
        [dst, jnp.arange(n_extra, dtype=jnp.int32) % N_PAD])
    src_idx = jnp.stack([src0, src1])
    dst_idx = dst_pad.reshape(N_CHUNKS, CHUNK)
    xh = jnp.concatenate(
        [x[:, :H], x[:, H:], jnp.zeros((8, H), jnp.float32)], axis=0)
    zeros = jnp.zeros((CHUNK, H), jnp.float32)

    agg_cat = _sc_segment_sum(xh, src_idx, dst_idx, zeros)
    z = _tc_root(x, W_root.T, b.reshape(1, D))

    out = _tc_finish(
        agg_cat[:N_NODES], agg_cat[N_PAD:N_PAD + N_NODES], z,
        W_rel[:, :H].T, W_rel[:, H:].T,
    )
    return out


# speedup vs baseline: 2.3687x; 2.3687x over previous
import functools

import jax
import jax.numpy as jnp
from jax import lax
from jax.experimental import pallas as pl
from jax.experimental.pallas import tpu as pltpu
from jax.experimental.pallas import tpu_sc as plsc

N_NODES = 10000
N_PAD = 10240
N_EDGES = 160000
D = 256
H = D // 2
CHUNK = 128
N_TILES = 16
K_PER_TILE = 80
N_CHUNKS = N_TILES * K_PER_TILE
E_PADDED = N_CHUNKS * CHUNK
ROWS_PER_TILE = N_PAD // N_TILES
ZROW = 2 * N_NODES


def _sc_segment_sum(xh, src_idx, dst_idx, zeros):
    mesh = plsc.VectorSubcoreMesh(core_axis_name="c", subcore_axis_name="s")

    @functools.partial(
        pl.kernel,
        out_type=jax.ShapeDtypeStruct((2 * N_PAD, H), jnp.float32),
        mesh=mesh,
        scratch_types=[
            pltpu.VMEM((CHUNK,), jnp.int32),
            pltpu.VMEM((CHUNK,), jnp.int32),
            pltpu.VMEM((CHUNK,), jnp.int32),
            pltpu.VMEM((CHUNK,), jnp.int32),
            pltpu.VMEM((CHUNK, H), jnp.float32),
            pltpu.VMEM((CHUNK, H), jnp.float32),
            pltpu.VMEM_SHARED((N_PAD, H), jnp.float32),
        ]
        + [pltpu.SemaphoreType.DMA] * 4,
    )
    def sc_kernel(xh_hbm, src_hbm, dst_hbm, zeros_hbm, out_hbm,
                  src0, src1, dst0, dst1, rows0, rows1, acc_sh,
                  sg0, sg1, ss0, ss1):
        c = lax.axis_index("c")
        s = lax.axis_index("s")
        slots = ((src0, dst0, rows0, sg0, ss0), (src1, dst1, rows1, sg1, ss1))

        def idx_load(i, sl):
            src_v, dst_v = slots[sl][0], slots[sl][1]
            pltpu.sync_copy(src_hbm.at[c, i], src_v)
            pltpu.sync_copy(dst_hbm.at[i], dst_v)

        def g_start(sl):
            src_v, rows_v, sem = slots[sl][0], slots[sl][2], slots[sl][3]
            pltpu.async_copy(xh_hbm.at[src_v], rows_v, sem)

        def g_wait(sl):
            src_v, rows_v, sem = slots[sl][0], slots[sl][2], slots[sl][3]
            pltpu.make_async_copy(xh_hbm.at[src_v], rows_v, sem).wait()

        def s_start(sl):
            dst_v, rows_v, sem = slots[sl][1], slots[sl][2], slots[sl][4]
            pltpu.async_copy(rows_v, acc_sh.at[dst_v], sem, add=True)

        def s_wait(sl):
            dst_v, rows_v, sem = slots[sl][1], slots[sl][2], slots[sl][4]
            pltpu.make_async_copy(rows_v, acc_sh.at[dst_v], sem).wait()

        idx_load(s, 0)
        g_start(0)
        pltpu.sync_copy(zeros_hbm, rows1)
        for z in range(ROWS_PER_TILE // CHUNK):
            pltpu.sync_copy(
                rows1, acc_sh.at[pl.ds(s * ROWS_PER_TILE + z * CHUNK, CHUNK)])
        plsc.subcore_barrier()

        idx_load(N_TILES + s, 1)
        g_wait(0)
        g_start(1)
        s_start(0)

        @pl.loop(0, (K_PER_TILE - 2) // 2)
        def _(g):
            for half in range(2):
                k = 2 * g + 1 + half
                cur, nxt = (1, 0) if half == 0 else (0, 1)
                s_wait(nxt)
                idx_load((k + 1) * N_TILES + s, nxt)
                g_wait(cur)
                g_start(nxt)
                s_start(cur)

        s_wait(0)
        g_wait(1)
        s_start(1)
        s_wait(1)

        plsc.subcore_barrier()
        pltpu.sync_copy(
            acc_sh.at[pl.ds(s * ROWS_PER_TILE, ROWS_PER_TILE)],
            out_hbm.at[pl.ds(c * N_PAD + s * ROWS_PER_TILE, ROWS_PER_TILE)],
        )

    return sc_kernel(xh, src_idx, dst_idx, zeros)


def _tc_root_body(x_ref, wr_ref, b_ref, o_ref):
    o_ref[...] = jnp.dot(x_ref[...], wr_ref[...],
                         preferred_element_type=jnp.float32) + b_ref[...]


def _tc_root(x, wr, b2d):
    n = x.shape[0]
    blk = 1000
    return pl.pallas_call(
        _tc_root_body,
        grid=(n // blk,),
        in_specs=[
            pl.BlockSpec((blk, D), lambda i: (i, 0)),
            pl.BlockSpec((D, D), lambda i: (0, 0)),
            pl.BlockSpec((1, D), lambda i: (0, 0)),
        ],
        out_specs=pl.BlockSpec((blk, D), lambda i: (i, 0)),
        out_shape=jax.ShapeDtypeStruct((n, D), jnp.float32),
    )(x, wr, b2d)


def _tc_finish_body(a0_ref, a1_ref, z_ref, w0_ref, w1_ref, o_ref):
    y = jnp.dot(a0_ref[...], w0_ref[...], preferred_element_type=jnp.float32)
    y = y + jnp.dot(a1_ref[...], w1_ref[...], preferred_element_type=jnp.float32)
    y = y + z_ref[...]
    m = jnp.max(y, axis=-1, keepdims=True)
    t = y - m
    lse = jnp.log(jnp.sum(jnp.exp(t), axis=-1, keepdims=True))
    o_ref[...] = t - lse


def _tc_finish(agg0, agg1, z, w0, w1):
    n = z.shape[0]
    blk = 1000
    return pl.pallas_call(
        _tc_finish_body,
        grid=(n // blk,),
        in_specs=[
            pl.BlockSpec((blk, H), lambda i: (i, 0)),
            pl.BlockSpec((blk, H), lambda i: (i, 0)),
            pl.BlockSpec((blk, D), lambda i: (i, 0)),
            pl.BlockSpec((H, D), lambda i: (0, 0)),
            pl.BlockSpec((H, D), lambda i: (0, 0)),
        ],
        out_specs=pl.BlockSpec((blk, D), lambda i: (i, 0)),
        out_shape=jax.ShapeDtypeStruct((n, D), jnp.float32),
    )(agg0, agg1, z, w0, w1)


def kernel(x, edge_index, W_rel, W_root, b):
    src = edge_index[0]
    dst = edge_index[1]
    n_extra = E_PADDED - N_EDGES
    pad_src = jnp.arange(n_extra, dtype=jnp.int32) * 37 % N_NODES
    src_p = jnp.concatenate([src, pad_src])
    src0 = src_p.reshape(N_CHUNKS, CHUNK)
    src1 = (src_p + N_NODES).reshape(N_CHUNKS, CHUNK)
    dst_pad = jnp.concatenate(
        [dst, N_NODES + (jnp.arange(n_extra, dtype=jnp.int32) % (N_PAD - N_NODES))])
    src_idx = jnp.stack([src0, src1])
    dst_idx = dst_pad.reshape(N_CHUNKS, CHUNK)
    xh = jnp.concatenate([x[:, :H], x[:, H:]], axis=0)
    zeros = jnp.zeros((CHUNK, H), jnp.float32)

    agg_cat = _sc_segment_sum(xh, src_idx, dst_idx, zeros)
    z = _tc_root(x, W_root.T, b.reshape(1, D))

    out = _tc_finish(
        agg_cat[:N_NODES], agg_cat[N_PAD:N_PAD + N_NODES], z,
        W_rel[:, :H].T, W_rel[:, H:].T,
    )
    return out


# speedup vs baseline: 2.9679x; 1.2530x over previous
import functools

import jax
import jax.numpy as jnp
from jax import lax
from jax.experimental import pallas as pl
from jax.experimental.pallas import tpu as pltpu
from jax.experimental.pallas import tpu_sc as plsc

N_NODES = 10000
N_PAD = 10240
N_EDGES = 160000
D = 256
H = D // 2
CHUNK = 128
N_TILES = 16
K_PER_TILE = 80
N_CHUNKS = N_TILES * K_PER_TILE
E_PADDED = N_CHUNKS * CHUNK
ROWS_PER_TILE = N_PAD // N_TILES
ZROW = 2 * N_NODES


def _sc_segment_sum(xh, src_idx, dst_idx, zeros):
    mesh = plsc.VectorSubcoreMesh(core_axis_name="c", subcore_axis_name="s")

    @functools.partial(
        pl.kernel,
        out_type=jax.ShapeDtypeStruct((2 * N_PAD, H), jnp.float32),
        mesh=mesh,
        scratch_types=[
            pltpu.VMEM((4, CHUNK), jnp.int32),
            pltpu.VMEM((4, CHUNK), jnp.int32),
            pltpu.VMEM((CHUNK, H), jnp.float32),
            pltpu.VMEM((CHUNK, H), jnp.float32),
            pltpu.VMEM_SHARED((N_PAD, H), jnp.float32),
        ]
        + [pltpu.SemaphoreType.DMA] * 8,
    )
    def sc_kernel(xh_hbm, src_hbm, dst_hbm, zeros_hbm, out_hbm,
                  srcq, dstq, rows0, rows1, acc_sh, *sems):
        sem_i = sems[:4]
        sem_g = sems[4:6]
        sem_s = sems[6:8]
        rows = (rows0, rows1)
        c = lax.axis_index("c")
        s = lax.axis_index("s")

        def i_start(k, q):
            i = k * N_TILES + s
            pltpu.async_copy(src_hbm.at[c, i], srcq.at[q], sem_i[q])
            pltpu.async_copy(dst_hbm.at[i], dstq.at[q], sem_i[q])

        def i_wait(k, q):
            i = k * N_TILES + s
            pltpu.make_async_copy(src_hbm.at[c, i], srcq.at[q], sem_i[q]).wait()
            pltpu.make_async_copy(dst_hbm.at[i], dstq.at[q], sem_i[q]).wait()

        def g_start(q, r):
            pltpu.async_copy(xh_hbm.at[srcq.at[q]], rows[r], sem_g[r])

        def g_wait(q, r):
            pltpu.make_async_copy(xh_hbm.at[srcq.at[q]], rows[r], sem_g[r]).wait()

        def s_start(q, r):
            pltpu.async_copy(rows[r], acc_sh.at[dstq.at[q]], sem_s[r], add=True)

        def s_wait(q, r):
            pltpu.make_async_copy(rows[r], acc_sh.at[dstq.at[q]], sem_s[r]).wait()

        def body(k, do_swait=True, do_iwg=True, do_istart=True):
            r = k % 2
            rb = 1 - r
            q = k % 4
            if do_swait:
                s_wait((k - 1) % 4, rb)
            if do_iwg:
                i_wait(k + 1, (k + 1) % 4)
                g_start((k + 1) % 4, rb)
            g_wait(q, r)
            s_start(q, r)
            if do_istart:
                i_start(k + 3, (k + 3) % 4)

        i_start(0, 0)
        i_start(1, 1)
        i_start(2, 2)
        i_wait(0, 0)
        g_start(0, 0)
        pltpu.sync_copy(zeros_hbm, rows1)
        for z in range(ROWS_PER_TILE // CHUNK):
            pltpu.sync_copy(
                rows1, acc_sh.at[pl.ds(s * ROWS_PER_TILE + z * CHUNK, CHUNK)])
        plsc.subcore_barrier()

        body(0, do_swait=False)
        body(1)

        @pl.loop(0, (K_PER_TILE - 8) // 4)
        def _(g):
            for jj in range(4):
                body_k = jj + 2
                k_dyn = g * 4 + body_k
                r = body_k % 2
                rb = 1 - r
                q = body_k % 4
                s_wait((body_k - 1) % 4, rb)
                i_wait(k_dyn + 1, (body_k + 1) % 4)
                g_start((body_k + 1) % 4, rb)
                g_wait(q, r)
                s_start(q, r)
                i_start(k_dyn + 3, (body_k + 3) % 4)

        body(74)
        body(75)
        body(76)
        body(77, do_istart=False)
        body(78, do_istart=False)
        body(79, do_iwg=False, do_istart=False)
        s_wait(79 % 4, 1)

        plsc.subcore_barrier()
        pltpu.sync_copy(
            acc_sh.at[pl.ds(s * ROWS_PER_TILE, ROWS_PER_TILE)],
            out_hbm.at[pl.ds(c * N_PAD + s * ROWS_PER_TILE, ROWS_PER_TILE)],
        )

    return sc_kernel(xh, src_idx, dst_idx, zeros)


def _tc_root_body(x_ref, wr_ref, b_ref, o_ref):
    o_ref[...] = jnp.dot(x_ref[...], wr_ref[...],
                         preferred_element_type=jnp.float32) + b_ref[...]


def _tc_root(x, wr, b2d):
    n = x.shape[0]
    blk = 1000
    return pl.pallas_call(
        _tc_root_body,
        grid=(n // blk,),
        in_specs=[
            pl.BlockSpec((blk, D), lambda i: (i, 0)),
            pl.BlockSpec((D, D), lambda i: (0, 0)),
            pl.BlockSpec((1, D), lambda i: (0, 0)),
        ],
        out_specs=pl.BlockSpec((blk, D), lambda i: (i, 0)),
        out_shape=jax.ShapeDtypeStruct((n, D), jnp.float32),
    )(x, wr, b2d)


def _tc_finish_body(a0_ref, a1_ref, z_ref, w0_ref, w1_ref, o_ref):
    y = jnp.dot(a0_ref[...], w0_ref[...], preferred_element_type=jnp.float32)
    y = y + jnp.dot(a1_ref[...], w1_ref[...], preferred_element_type=jnp.float32)
    y = y + z_ref[...]
    m = jnp.max(y, axis=-1, keepdims=True)
    t = y - m
    lse = jnp.log(jnp.sum(jnp.exp(t), axis=-1, keepdims=True))
    o_ref[...] = t - lse


def _tc_finish(agg0, agg1, z, w0, w1):
    n = z.shape[0]
    blk = 1000
    return pl.pallas_call(
        _tc_finish_body,
        grid=(n // blk,),
        in_specs=[
            pl.BlockSpec((blk, H), lambda i: (i, 0)),
            pl.BlockSpec((blk, H), lambda i: (i, 0)),
            pl.BlockSpec((blk, D), lambda i: (i, 0)),
            pl.BlockSpec((H, D), lambda i: (0, 0)),
            pl.BlockSpec((H, D), lambda i: (0, 0)),
        ],
        out_specs=pl.BlockSpec((blk, D), lambda i: (i, 0)),
        out_shape=jax.ShapeDtypeStruct((n, D), jnp.float32),
    )(agg0, agg1, z, w0, w1)


def kernel(x, edge_index, W_rel, W_root, b):
    src = edge_index[0]
    dst = edge_index[1]
    n_extra = E_PADDED - N_EDGES
    pad_src = jnp.arange(n_extra, dtype=jnp.int32) * 37 % N_NODES
    src_p = jnp.concatenate([src, pad_src])
    src0 = src_p.reshape(N_CHUNKS, CHUNK)
    src1 = (src_p + N_NODES).reshape(N_CHUNKS, CHUNK)
    dst_pad = jnp.concatenate(
        [dst, N_NODES + (jnp.arange(n_extra, dtype=jnp.int32) % (N_PAD - N_NODES))])
    src_idx = jnp.stack([src0, src1])
    dst_idx = dst_pad.reshape(N_CHUNKS, CHUNK)
    xh = jnp.concatenate([x[:, :H], x[:, H:]], axis=0)
    zeros = jnp.zeros((CHUNK, H), jnp.float32)

    agg_cat = _sc_segment_sum(xh, src_idx, dst_idx, zeros)
    z = _tc_root(x, W_root.T, b.reshape(1, D))

    out = _tc_finish(
        agg_cat[:N_NODES], agg_cat[N_PAD:N_PAD + N_NODES], z,
        W_rel[:, :H].T, W_rel[:, H:].T,
    )
    return out


# speedup vs baseline: 3.0069x; 1.0131x over previous
import functools

import jax
import jax.numpy as jnp
from jax import lax
from jax.experimental import pallas as pl
from jax.experimental.pallas import tpu as pltpu
from jax.experimental.pallas import tpu_sc as plsc

N_NODES = 10000
N_PAD = 10240
N_EDGES = 160000
D = 256
H = D // 2
CHUNK = 128
N_TILES = 16
K_PER_TILE = 80
N_CHUNKS = N_TILES * K_PER_TILE
E_PADDED = N_CHUNKS * CHUNK
ROWS_PER_TILE = N_PAD // N_TILES
ZROW = 2 * N_NODES


def _sc_segment_sum(xh, src_idx, dst_idx, zeros):
    mesh = plsc.VectorSubcoreMesh(core_axis_name="c", subcore_axis_name="s")

    @functools.partial(
        pl.kernel,
        out_type=jax.ShapeDtypeStruct((2 * N_PAD, H), jnp.float32),
        mesh=mesh,
        scratch_types=[
            pltpu.VMEM((4, CHUNK), jnp.int32),
            pltpu.VMEM((4, CHUNK), jnp.int32),
            pltpu.VMEM((CHUNK, H), jnp.float32),
            pltpu.VMEM((CHUNK, H), jnp.float32),
            pltpu.VMEM_SHARED((N_PAD, H), jnp.float32),
        ]
        + [pltpu.SemaphoreType.DMA] * 8,
    )
    def sc_kernel(xh_hbm, src_hbm, dst_hbm, zeros_hbm, out_hbm,
                  srcq, dstq, rows0, rows1, acc_sh, *sems):
        sem_i = sems[:4]
        sem_g = sems[4:6]
        sem_s = sems[6:8]
        rows = (rows0, rows1)
        c = lax.axis_index("c")
        s = lax.axis_index("s")

        def i_start(k, q):
            i = k * N_TILES + s
            pltpu.async_copy(src_hbm.at[c, i], srcq.at[q], sem_i[q])
            pltpu.async_copy(dst_hbm.at[i], dstq.at[q], sem_i[q])

        def i_wait(k, q):
            i = k * N_TILES + s
            pltpu.make_async_copy(src_hbm.at[c, i], srcq.at[q], sem_i[q]).wait()
            pltpu.make_async_copy(dst_hbm.at[i], dstq.at[q], sem_i[q]).wait()

        def g_start(q, r):
            pltpu.async_copy(xh_hbm.at[srcq.at[q]], rows[r], sem_g[r])

        def g_wait(q, r):
            pltpu.make_async_copy(xh_hbm.at[srcq.at[q]], rows[r], sem_g[r]).wait()

        def s_start(q, r):
            pltpu.async_copy(rows[r], acc_sh.at[dstq.at[q]], sem_s[r], add=True)

        def s_wait(q, r):
            pltpu.make_async_copy(rows[r], acc_sh.at[dstq.at[q]], sem_s[r]).wait()

        def body(k, do_swait=True, do_iwg=True, do_istart=True):
            r = k % 2
            rb = 1 - r
            q = k % 4
            if do_swait:
                s_wait((k - 1) % 4, rb)
            if do_iwg:
                i_wait(k + 1, (k + 1) % 4)
                g_start((k + 1) % 4, rb)
            g_wait(q, r)
            s_start(q, r)
            if do_istart:
                i_start(k + 3, (k + 3) % 4)

        i_start(0, 0)
        i_start(1, 1)
        i_start(2, 2)
        i_wait(0, 0)
        g_start(0, 0)
        pltpu.sync_copy(zeros_hbm, rows1)
        for z in range(ROWS_PER_TILE // CHUNK):
            pltpu.sync_copy(
                rows1, acc_sh.at[pl.ds(s * ROWS_PER_TILE + z * CHUNK, CHUNK)])
        plsc.subcore_barrier()

        body(0, do_swait=False)
        body(1)

        @pl.loop(0, (K_PER_TILE - 8) // 4)
        def _(g):
            for jj in range(4):
                body_k = jj + 2
                k_dyn = g * 4 + body_k
                r = body_k % 2
                rb = 1 - r
                q = body_k % 4
                s_wait((body_k - 1) % 4, rb)
                i_wait(k_dyn + 1, (body_k + 1) % 4)
                g_start((body_k + 1) % 4, rb)
                g_wait(q, r)
                s_start(q, r)
                i_start(k_dyn + 3, (body_k + 3) % 4)

        body(74)
        body(75)
        body(76)
        body(77, do_istart=False)
        body(78, do_istart=False)
        body(79, do_iwg=False, do_istart=False)
        s_wait(79 % 4, 1)

        plsc.subcore_barrier()
        pltpu.sync_copy(
            acc_sh.at[pl.ds(s * ROWS_PER_TILE, ROWS_PER_TILE)],
            out_hbm.at[pl.ds(c * N_PAD + s * ROWS_PER_TILE, ROWS_PER_TILE)],
        )

    return sc_kernel(xh, src_idx, dst_idx, zeros)


def _tc_root_body(x_ref, wr_ref, b_ref, o_ref):
    o_ref[...] = jnp.dot(x_ref[...], wr_ref[...],
                         preferred_element_type=jnp.float32) + b_ref[...]


def _tc_root(x, wr, b2d):
    n = x.shape[0]
    blk = 1000
    return pl.pallas_call(
        _tc_root_body,
        grid=(n // blk,),
        in_specs=[
            pl.BlockSpec((blk, D), lambda i: (i, 0)),
            pl.BlockSpec((D, D), lambda i: (0, 0)),
            pl.BlockSpec((1, D), lambda i: (0, 0)),
        ],
        out_specs=pl.BlockSpec((blk, D), lambda i: (i, 0)),
        out_shape=jax.ShapeDtypeStruct((n, D), jnp.float32),
    )(x, wr, b2d)


def _tc_finish_body(a0_ref, a1_ref, x_ref, w0_ref, w1_ref, wr_ref, b_ref, o_ref):
    y = jnp.dot(a0_ref[...], w0_ref[...], preferred_element_type=jnp.float32)
    y = y + jnp.dot(a1_ref[...], w1_ref[...], preferred_element_type=jnp.float32)
    y = y + jnp.dot(x_ref[...], wr_ref[...], preferred_element_type=jnp.float32)
    y = y + b_ref[...]
    m = jnp.max(y, axis=-1, keepdims=True)
    t = y - m
    lse = jnp.log(jnp.sum(jnp.exp(t), axis=-1, keepdims=True))
    o_ref[...] = t - lse


def _tc_finish(agg0, agg1, x, w0, w1, wr, b2d):
    n = x.shape[0]
    blk = 1000
    return pl.pallas_call(
        _tc_finish_body,
        grid=(n // blk,),
        in_specs=[
            pl.BlockSpec((blk, H), lambda i: (i, 0)),
            pl.BlockSpec((blk, H), lambda i: (i, 0)),
            pl.BlockSpec((blk, D), lambda i: (i, 0)),
            pl.BlockSpec((H, D), lambda i: (0, 0)),
            pl.BlockSpec((H, D), lambda i: (0, 0)),
            pl.BlockSpec((D, D), lambda i: (0, 0)),
            pl.BlockSpec((1, D), lambda i: (0, 0)),
        ],
        out_specs=pl.BlockSpec((blk, D), lambda i: (i, 0)),
        out_shape=jax.ShapeDtypeStruct((n, D), jnp.float32),
    )(agg0, agg1, x, w0, w1, wr, b2d)


def kernel(x, edge_index, W_rel, W_root, b):
    src = edge_index[0]
    dst = edge_index[1]
    n_extra = E_PADDED - N_EDGES
    pad_src = jnp.arange(n_extra, dtype=jnp.int32) * 37 % N_NODES
    src_p = jnp.concatenate([src, pad_src])
    src0 = src_p.reshape(N_CHUNKS, CHUNK)
    src1 = (src_p + N_NODES).reshape(N_CHUNKS, CHUNK)
    dst_pad = jnp.concatenate(
        [dst, N_NODES + (jnp.arange(n_extra, dtype=jnp.int32) % (N_PAD - N_NODES))])
    src_idx = jnp.stack([src0, src1])
    dst_idx = dst_pad.reshape(N_CHUNKS, CHUNK)
    xh = jnp.concatenate([x[:, :H], x[:, H:]], axis=0)
    zeros = jnp.zeros((CHUNK, H), jnp.float32)

    agg_cat = _sc_segment_sum(xh, src_idx, dst_idx, zeros)

    out = _tc_finish(
        agg_cat[:N_NODES], agg_cat[N_PAD:N_PAD + N_NODES], x,
        W_rel[:, :H].T, W_rel[:, H:].T, W_root.T, b.reshape(1, D),
    )
    return out
